# baseline (speedup 1.0000x reference)
import functools

import jax
import jax.numpy as jnp
from jax import lax
from jax.experimental import pallas as pl
from jax.experimental.pallas import tpu as pltpu
from jax.experimental.pallas import tpu_sc as plsc

_D = 1024
_E = 8
_LANES = 16
_KC = _D // _LANES
_TSUB = 4
_NW = 32
_TC_BLOCK = 1024
_N_SC = 1024


def _router_block(x_ref, wt_ref, b_ref, out_ref):
    logits = jax.lax.dot_general(
        x_ref[...], wt_ref[...], (((1,), (0,)), ((), ())),
        precision=jax.lax.Precision.DEFAULT,
        preferred_element_type=jnp.float32)
    logits = logits + b_ref[...]
    m = jnp.max(logits, axis=-1, keepdims=True)
    e = jnp.exp(logits - m)
    out_ref[...] = e / jnp.sum(e, axis=-1, keepdims=True)


def _tc_router(x, Wt, b, n_tc):
    return pl.pallas_call(
        _router_block,
        grid=(n_tc // _TC_BLOCK,),
        in_specs=[
            pl.BlockSpec((_TC_BLOCK, _D), lambda i: (i, 0)),
            pl.BlockSpec((_D, _E), lambda i: (0, 0)),
            pl.BlockSpec((1, _E), lambda i: (0, 0)),
        ],
        out_specs=pl.BlockSpec((_TC_BLOCK, _E), lambda i: (i, 0)),
        out_shape=jax.ShapeDtypeStruct((n_tc, _E), jnp.float32),
        compiler_params=pltpu.CompilerParams(
            dimension_semantics=("arbitrary",),
        ),
    )(x, Wt, b.reshape(1, _E))


def _sc_router_body(base_tok, x_hbm, w_hbm, b2_hbm, out_hbm,
                    w_v, b_v, x_v, out_v):
    nc = 2
    wid = lax.axis_index("s") * nc + lax.axis_index("c")
    tok_per_w = _N_SC // _NW
    nblk = tok_per_w // _LANES
    base = base_tok + wid * tok_per_w

    pltpu.sync_copy(w_hbm, w_v)
    pltpu.sync_copy(b2_hbm, b_v)

    tok_iota = lax.iota(jnp.int32, _LANES)
    b2 = b_v[...]

    def blk_body(blk, _):
        tok0 = base + blk * _LANES
        pltpu.sync_copy(x_hbm.at[pl.ds(tok0, _LANES)], x_v)

        svals = [[None] * _E for _ in range(_LANES)]
        for sub in range(_LANES // _TSUB):
            def kc_body(kc, accs):
                accs = list(accs)
                for t in range(_TSUB):
                    xv = x_v[sub * _TSUB + t, pl.ds(kc * _LANES, _LANES)]
                    for e in range(_E):
                        wv = w_v[e, pl.ds(kc * _LANES, _LANES)]
                        accs[t * _E + e] = accs[t * _E + e] + xv * wv
                return tuple(accs)

            init = tuple(jnp.zeros((_LANES,), jnp.float32)
                         for _ in range(_TSUB * _E))
            accs = lax.fori_loop(0, _KC, kc_body, init)
            for t in range(_TSUB):
                for e in range(_E):
                    svals[sub * _TSUB + t][e] = jnp.sum(accs[t * _E + e])

        for p in range(_LANES // 2):
            v = jnp.zeros((_LANES,), jnp.float32)
            for e in range(_E):
                v = jnp.where(tok_iota == e, svals[2 * p][e], v)
                v = jnp.where(tok_iota == e + _E, svals[2 * p + 1][e], v)
            ev = jnp.exp(v + b2)
            cs = plsc.cumsum(ev)
            s0 = cs[_E - 1]
            s1 = cs[_LANES - 1] - s0
            out_v[p, :] = ev / jnp.where(tok_iota < _E, s0, s1)

        row0 = pl.multiple_of((tok0 - base_tok) // 2, _LANES // 2)
        pltpu.sync_copy(out_v, out_hbm.at[pl.ds(row0, _LANES // 2)])
        return ()

    lax.fori_loop(0, nblk, blk_body, ())


def _sc_router(x, W, b2, base_tok):
    mesh = plsc.VectorSubcoreMesh(core_axis_name="c", subcore_axis_name="s")
    f = functools.partial(
        pl.kernel,
        mesh=mesh,
        out_type=jax.ShapeDtypeStruct((_N_SC // 2, _LANES), jnp.float32),
        scratch_types=[
            pltpu.VMEM((_E, _D), jnp.float32),
            pltpu.VMEM((_LANES,), jnp.float32),
            pltpu.VMEM((_LANES, _D), jnp.float32),
            pltpu.VMEM((_LANES // 2, _LANES), jnp.float32),
        ],
        compiler_params=pltpu.CompilerParams(needs_layout_passes=False),
    )(functools.partial(_sc_router_body, base_tok))
    return f(x, W, b2)


def kernel(x, W, b):
    n = x.shape[0]
    n_tc = n - _N_SC
    out_tc = _tc_router(x, W.T, b, n_tc)
    b2 = jnp.concatenate([b, b])
    out_sc = _sc_router(x, W, b2, n_tc).reshape(_N_SC, _E)
    return jnp.concatenate([out_tc, out_sc], axis=0)


# speedup vs baseline: 1.4954x; 1.4954x over previous
import jax
import jax.numpy as jnp
from jax.experimental import pallas as pl
from jax.experimental.pallas import tpu as pltpu

_BLOCK = 2048


def _router_block(x_ref, wt_ref, b_ref, out_ref):
    xb = x_ref[...].astype(jnp.bfloat16)
    logits = jnp.dot(xb, wt_ref[...], preferred_element_type=jnp.float32)
    logits = logits + b_ref[...]
    m = jnp.max(logits, axis=-1, keepdims=True)
    e = jnp.exp(logits - m)
    out_ref[...] = e / jnp.sum(e, axis=-1, keepdims=True)


def kernel(x, W, b):
    n_tokens, d = x.shape
    n_experts = W.shape[0]
    grid = (n_tokens // _BLOCK,)
    return pl.pallas_call(
        _router_block,
        grid=grid,
        in_specs=[
            pl.BlockSpec((_BLOCK, d), lambda i: (i, 0)),
            pl.BlockSpec((d, n_experts), lambda i: (0, 0)),
            pl.BlockSpec((1, n_experts), lambda i: (0, 0)),
        ],
        out_specs=pl.BlockSpec((_BLOCK, n_experts), lambda i: (i, 0)),
        out_shape=jax.ShapeDtypeStruct((n_tokens, n_experts), jnp.float32),
        compiler_params=pltpu.CompilerParams(
            dimension_semantics=("arbitrary",),
        ),
    )(x, W.T.astype(jnp.bfloat16), b.reshape(1, n_experts))
